# baseline (speedup 1.0000x reference)
import functools

import jax
import jax.numpy as jnp
from jax import lax
from jax.experimental import pallas as pl
from jax.experimental.pallas import tpu as pltpu
from jax.experimental.pallas import tpu_sc as plsc

NS = 16
NC = 2
EK = 96
NK = 128
NPAD = NS * 5 * NK


def _zero_fill(buf, nrows):
    @pl.loop(0, nrows * 8)
    def _(t):
        buf[t // 8, pl.ds((t % 8) * 16, 16)] = jnp.zeros((16,), jnp.float32)


def _zero_acc_stripe(zbuf, acc, s, rows_per_tile):
    zr = zbuf.shape[0]
    nch = rows_per_tile // zr
    for k in range(nch):
        pltpu.sync_copy(zbuf, acc.at[pl.ds(s * rows_per_tile + k * zr, zr)])


def _sc_scatter_rows(featA, featB, idx_r, n_out):
    n = featA.shape[0]
    rpt_src = n // NS
    nch = rpt_src // NK
    rpt_out = n_out // NS
    mesh = plsc.VectorSubcoreMesh(core_axis_name="c", subcore_axis_name="s")

    @functools.partial(
        pl.kernel,
        out_type=[jax.ShapeDtypeStruct((n_out, 128), jnp.float32)] * 2,
        mesh=mesh,
        scratch_types=[
            pltpu.VMEM((nch, NK), jnp.int32),
            pltpu.VMEM((NK, 128), jnp.float32),
            pltpu.VMEM_SHARED((n_out, 128), jnp.float32),
            pltpu.SemaphoreType.DMA,
        ],
    )
    def body(fA, fB, idx_hbm, oA, oB, idx_v, rows_v, acc, sem):
        c = lax.axis_index("c")
        s = lax.axis_index("s")
        _zero_fill(rows_v, NK)
        _zero_acc_stripe(rows_v, acc, s, rpt_out)
        pltpu.sync_copy(idx_hbm.at[s], idx_v)
        plsc.subcore_barrier()

        @pl.loop(0, nch)
        def _(j):
            @pl.when(c == 0)
            def _():
                pltpu.async_copy(
                    fA.at[pl.ds(s * rpt_src + j * NK, NK)], rows_v, sem
                ).wait()

            @pl.when(c == 1)
            def _():
                pltpu.async_copy(
                    fB.at[pl.ds(s * rpt_src + j * NK, NK)], rows_v, sem
                ).wait()

            pltpu.sync_copy(rows_v, acc.at[idx_v.at[j]], add=True)

        plsc.subcore_barrier()
        stripe = pl.ds(s * rpt_out, rpt_out)

        @pl.when(c == 0)
        def _():
            pltpu.sync_copy(acc.at[stripe], oA.at[stripe])

        @pl.when(c == 1)
        def _():
            pltpu.sync_copy(acc.at[stripe], oB.at[stripe])

    return body(featA, featB, idx_r)


def _sc_segsum2(hA, hB, cA, cB, pidx_r):
    n = hA.shape[0]
    nch_e = pidx_r.shape[1]
    rpt = n // NS
    mesh = plsc.VectorSubcoreMesh(core_axis_name="c", subcore_axis_name="s")

    @functools.partial(
        pl.kernel,
        out_type=[jax.ShapeDtypeStruct((n, 128), jnp.float32)] * 4,
        mesh=mesh,
        scratch_types=[
            pltpu.VMEM((nch_e, EK), jnp.int32),
            pltpu.VMEM((EK, 128), jnp.float32),
            pltpu.VMEM((EK, 128), jnp.float32),
            pltpu.VMEM((EK,), jnp.int32),
            pltpu.VMEM((EK,), jnp.int32),
            pltpu.VMEM((EK,), jnp.int32),
            pltpu.VMEM((EK,), jnp.int32),
            pltpu.VMEM_SHARED((n, 128), jnp.float32),
            pltpu.SemaphoreType.DMA,
            pltpu.SemaphoreType.DMA,
        ],
    )
    def body(hA_h, hB_h, cA_h, cB_h, pidx_h, oHA, oHB, oCA, oCB,
             pidx_v, rows0, rows1, src0, dst0, src1, dst1, acc, s0, s1):
        c = lax.axis_index("c")
        s = lax.axis_index("s")
        pltpu.sync_copy(pidx_h.at[s], pidx_v)
        stripe = pl.ds(s * rpt, rpt)

        offs = [16 * q for q in range(EK // 16)] + ([EK - 16] if EK % 16 else [])

        def unpack(j, src_b, dst_b):
            for off in offs:
                v = pidx_v[j, pl.ds(off, 16)]
                src_b[pl.ds(off, 16)] = v & 16383
                dst_b[pl.ds(off, 16)] = lax.shift_right_logical(v, 14)

        for tabA, tabB, outA, outB in ((hA_h, hB_h, oHA, oHB),
                                       (cA_h, cB_h, oCA, oCB)):
            _zero_fill(rows0, EK)
            _zero_acc_stripe(rows0, acc, s, rpt)
            plsc.subcore_barrier()

            def gather(src_b, buf, sem):
                @pl.when(c == 0)
                def _():
                    pltpu.async_copy(tabA.at[src_b], buf, sem)

                @pl.when(c == 1)
                def _():
                    pltpu.async_copy(tabB.at[src_b], buf, sem)

            def gwait(src_b, buf, sem):
                pltpu.make_async_copy(tabA.at[src_b], buf, sem).wait()

            unpack(0, src0, dst0)
            gather(src0, rows0, s0)

            n_pairs = (nch_e - 1) // 2 if nch_e % 2 else nch_e // 2 - 1

            @pl.loop(0, n_pairs)
            def _(p):
                j0 = 2 * p
                unpack(j0 + 1, src1, dst1)
                gwait(src0, rows0, s0)
                gather(src1, rows1, s1)
                pltpu.sync_copy(rows0, acc.at[dst0], add=True)
                unpack(j0 + 2, src0, dst0)
                gather(src0, rows0, s0)
                gwait(src1, rows1, s1)
                pltpu.sync_copy(rows1, acc.at[dst1], add=True)

            if nch_e % 2:
                gwait(src0, rows0, s0)
                pltpu.sync_copy(rows0, acc.at[dst0], add=True)
            else:
                unpack(nch_e - 1, src1, dst1)
                gwait(src0, rows0, s0)
                gather(src1, rows1, s1)
                pltpu.sync_copy(rows0, acc.at[dst0], add=True)
                gwait(src1, rows1, s1)
                pltpu.sync_copy(rows1, acc.at[dst1], add=True)

            plsc.subcore_barrier()

            @pl.when(c == 0)
            def _():
                pltpu.sync_copy(acc.at[stripe], outA.at[stripe])

            @pl.when(c == 1)
            def _():
                pltpu.sync_copy(acc.at[stripe], outB.at[stripe])

            plsc.subcore_barrier()

    return body(hA, hB, cA, cB, pidx_r)


def _sc_gather_rows(hA, hB, idx_r):
    n = hA.shape[0]
    nch = idx_r.shape[1]
    n_out = NS * nch * NK
    rpt = n_out // NS
    mesh = plsc.VectorSubcoreMesh(core_axis_name="c", subcore_axis_name="s")

    @functools.partial(
        pl.kernel,
        out_type=[jax.ShapeDtypeStruct((n_out, 128), jnp.float32)] * 2,
        mesh=mesh,
        scratch_types=[
            pltpu.VMEM((nch, NK), jnp.int32),
            pltpu.VMEM((NK, 128), jnp.float32),
            pltpu.SemaphoreType.DMA,
        ],
    )
    def body(hA_h, hB_h, idx_hbm, oA, oB, idx_v, rows_v, sem):
        c = lax.axis_index("c")
        s = lax.axis_index("s")
        pltpu.sync_copy(idx_hbm.at[s], idx_v)

        @pl.loop(0, nch)
        def _(j):
            @pl.when(c == 0)
            def _():
                pltpu.async_copy(hA_h.at[idx_v.at[j]], rows_v, sem).wait()
                pltpu.sync_copy(rows_v, oA.at[pl.ds(s * rpt + j * NK, NK)])

            @pl.when(c == 1)
            def _():
                pltpu.async_copy(hB_h.at[idx_v.at[j]], rows_v, sem).wait()
                pltpu.sync_copy(rows_v, oB.at[pl.ds(s * rpt + j * NK, NK)])

    return body(hA, hB, idx_r)


def _tc_wxh(xA, xB, Wx, b, bn):
    n = xA.shape[0]
    k, fo = Wx.shape

    def mm_body(xa_ref, xb_ref, w_ref, b_ref, o_ref):
        x = jnp.concatenate([xa_ref[...], xb_ref[...]], axis=1)
        o_ref[...] = (
            jnp.dot(x, w_ref[...], preferred_element_type=jnp.float32)
            + b_ref[...]
        )

    return pl.pallas_call(
        mm_body,
        grid=(n // bn,),
        in_specs=[
            pl.BlockSpec((bn, k // 2), lambda i: (i, 0)),
            pl.BlockSpec((bn, k // 2), lambda i: (i, 0)),
            pl.BlockSpec((k, fo), lambda i: (0, 0)),
            pl.BlockSpec((1, fo), lambda i: (0, 0)),
        ],
        out_specs=pl.BlockSpec((bn, fo), lambda i: (i, 0)),
        out_shape=jax.ShapeDtypeStruct((n, fo), jnp.float32),
    )(xA, xB, Wx, b.reshape(1, fo))


def _lstm_tail(g, c_agg, h_dim):
    i = g[:, 0:h_dim]
    f = g[:, h_dim:2 * h_dim]
    o = g[:, 2 * h_dim:3 * h_dim]
    gg = g[:, 3 * h_dim:4 * h_dim]
    cc = jax.nn.sigmoid(i) * jnp.tanh(gg)
    if c_agg is not None:
        cc += jax.nn.sigmoid(f) * c_agg
    hh = jax.nn.sigmoid(o) * jnp.tanh(cc)
    return hh, cc


def _tc_step0(wxh, h_dim, bn):
    n = wxh.shape[0]
    hh = h_dim // 2

    def body(wxh_ref, hA_o, hB_o, cA_o, cB_o):
        h, c = _lstm_tail(wxh_ref[...], None, h_dim)
        hA_o[...] = h[:, :hh]
        hB_o[...] = h[:, hh:]
        cA_o[...] = c[:, :hh]
        cB_o[...] = c[:, hh:]

    return pl.pallas_call(
        body,
        grid=(n // bn,),
        in_specs=[pl.BlockSpec((bn, 4 * h_dim), lambda i: (i, 0))],
        out_specs=[pl.BlockSpec((bn, hh), lambda i: (i, 0))] * 4,
        out_shape=[jax.ShapeDtypeStruct((n, hh), jnp.float32)] * 4,
    )(wxh)


def _tc_step(wxh, haA, haB, caA, caB, Wh, h_dim, bn):
    n = wxh.shape[0]
    hh = h_dim // 2

    def body(wxh_ref, haA_r, haB_r, caA_r, caB_r, wh_ref,
             hA_o, hB_o, cA_o, cB_o):
        h_agg = jnp.concatenate([haA_r[...], haB_r[...]], axis=1)
        g = wxh_ref[...] + jnp.dot(
            h_agg, wh_ref[...], preferred_element_type=jnp.float32
        )
        c_agg = jnp.concatenate([caA_r[...], caB_r[...]], axis=1)
        h, c = _lstm_tail(g, c_agg, h_dim)
        hA_o[...] = h[:, :hh]
        hB_o[...] = h[:, hh:]
        cA_o[...] = c[:, :hh]
        cB_o[...] = c[:, hh:]

    return pl.pallas_call(
        body,
        grid=(n // bn,),
        in_specs=[
            pl.BlockSpec((bn, 4 * h_dim), lambda i: (i, 0)),
            pl.BlockSpec((bn, hh), lambda i: (i, 0)),
            pl.BlockSpec((bn, hh), lambda i: (i, 0)),
            pl.BlockSpec((bn, hh), lambda i: (i, 0)),
            pl.BlockSpec((bn, hh), lambda i: (i, 0)),
            pl.BlockSpec((h_dim, 4 * h_dim), lambda i: (0, 0)),
        ],
        out_specs=[pl.BlockSpec((bn, hh), lambda i: (i, 0))] * 4,
        out_shape=[jax.ShapeDtypeStruct((n, hh), jnp.float32)] * 4,
    )(wxh, haA, haB, caA, caB, Wh)


def kernel(features, weights_x, weights_h, biases, index_map, edge_src,
           edge_dst):
    n, d = features.shape
    h_dim = weights_h.shape[1]
    n_layers = weights_h.shape[0]
    n_steps = 4
    bn = 512

    npad = NPAD
    pad_ids = jnp.arange(n, npad, dtype=jnp.int32)
    idx_p = jnp.concatenate([index_map.astype(jnp.int32), pad_ids])
    idx_r = idx_p.reshape(NS, npad // NS // NK, NK)
    e = edge_src.shape[0]
    pidx = edge_src.astype(jnp.int32) | (edge_dst.astype(jnp.int32) << 14)
    ept = e // NS
    ept_pad = -(-ept // EK) * EK
    pr = pidx.reshape(NS, ept)
    if ept_pad != ept:
        dummy = n + jnp.arange(NS, dtype=jnp.int32)
        dummy = (dummy | (dummy << 14))[:, None]
        pr = jnp.concatenate(
            [pr, jnp.broadcast_to(dummy, (NS, ept_pad - ept))], axis=1)
    pidx_r = pr.reshape(NS, ept_pad // EK, EK)

    zpad = jnp.zeros((npad - n, d // 2), jnp.float32)
    featA = jnp.concatenate([features[:, : d // 2], zpad])
    featB = jnp.concatenate([features[:, d // 2:], zpad])
    hA, hB = _sc_scatter_rows(featA, featB, idx_r, npad)

    start = 0
    for l in range(n_layers):
        in_dim = d if l == 0 else h_dim
        Wx = weights_x[start:start + in_dim]
        start += in_dim
        wxh = _tc_wxh(hA, hB, Wx, biases[l], bn)
        hA, hB, cA, cB = _tc_step0(wxh, h_dim, bn)
        for _ in range(n_steps - 1):
            haA, haB, caA, caB = _sc_segsum2(hA, hB, cA, cB, pidx_r)
            hA, hB, cA, cB = _tc_step(wxh, haA, haB, caA, caB,
                                      weights_h[l], h_dim, bn)

    oA, oB = _sc_gather_rows(hA, hB, idx_r)
    return jnp.concatenate([oA[:n], oB[:n]], axis=1)


# speedup vs baseline: 1.2283x; 1.2283x over previous
import functools

import jax
import jax.numpy as jnp
from jax import lax
from jax.experimental import pallas as pl
from jax.experimental.pallas import tpu as pltpu
from jax.experimental.pallas import tpu_sc as plsc

NS = 16
NC = 2
EK = 80
NK = 128
NPAD = NS * 5 * NK


def _zero_fill(buf, nrows):
    @pl.loop(0, nrows * 8)
    def _(t):
        buf[t // 8, pl.ds((t % 8) * 16, 16)] = jnp.zeros((16,), jnp.float32)


def _zero_acc_stripe(zbuf, acc, s, rows_per_tile):
    zr = zbuf.shape[0]
    nch = rows_per_tile // zr
    for k in range(nch):
        pltpu.sync_copy(zbuf, acc.at[pl.ds(s * rows_per_tile + k * zr, zr)])


def _sc_scatter_rows(featA, featB, idx_r, n_out):
    n = featA.shape[0]
    rpt_src = n // NS
    nch = rpt_src // NK
    rpt_out = n_out // NS
    mesh = plsc.VectorSubcoreMesh(core_axis_name="c", subcore_axis_name="s")

    @functools.partial(
        pl.kernel,
        out_type=[jax.ShapeDtypeStruct((n_out, 128), jnp.float32)] * 2,
        mesh=mesh,
        scratch_types=[
            pltpu.VMEM((nch, NK), jnp.int32),
            pltpu.VMEM((NK, 128), jnp.float32),
            pltpu.VMEM_SHARED((n_out, 128), jnp.float32),
            pltpu.SemaphoreType.DMA,
        ],
    )
    def body(fA, fB, idx_hbm, oA, oB, idx_v, rows_v, acc, sem):
        c = lax.axis_index("c")
        s = lax.axis_index("s")
        _zero_fill(rows_v, NK)
        _zero_acc_stripe(rows_v, acc, s, rpt_out)
        pltpu.sync_copy(idx_hbm.at[s], idx_v)
        plsc.subcore_barrier()

        @pl.loop(0, nch)
        def _(j):
            @pl.when(c == 0)
            def _():
                pltpu.async_copy(
                    fA.at[pl.ds(s * rpt_src + j * NK, NK)], rows_v, sem
                ).wait()

            @pl.when(c == 1)
            def _():
                pltpu.async_copy(
                    fB.at[pl.ds(s * rpt_src + j * NK, NK)], rows_v, sem
                ).wait()

            pltpu.sync_copy(rows_v, acc.at[idx_v.at[j]], add=True)

        plsc.subcore_barrier()
        stripe = pl.ds(s * rpt_out, rpt_out)

        @pl.when(c == 0)
        def _():
            pltpu.sync_copy(acc.at[stripe], oA.at[stripe])

        @pl.when(c == 1)
        def _():
            pltpu.sync_copy(acc.at[stripe], oB.at[stripe])

    return body(featA, featB, idx_r)


def _sc_segsum2(hA, hB, cA, cB, pidx_r):
    n = hA.shape[0]
    nch_e = pidx_r.shape[1]
    rpt = n // NS
    mesh = plsc.VectorSubcoreMesh(core_axis_name="c", subcore_axis_name="s")

    nbuf = 3
    trips = -(-nch_e // nbuf)

    @functools.partial(
        pl.kernel,
        out_type=[jax.ShapeDtypeStruct((n, 128), jnp.float32)] * 4,
        mesh=mesh,
        scratch_types=[
            pltpu.VMEM((nch_e, EK), jnp.int32),
            pltpu.VMEM((EK, 128), jnp.float32),
            pltpu.VMEM((EK, 128), jnp.float32),
            pltpu.VMEM((EK, 128), jnp.float32),
            pltpu.VMEM((EK,), jnp.int32),
            pltpu.VMEM((EK,), jnp.int32),
            pltpu.VMEM((EK,), jnp.int32),
            pltpu.VMEM((EK,), jnp.int32),
            pltpu.VMEM((EK,), jnp.int32),
            pltpu.VMEM((EK,), jnp.int32),
            pltpu.VMEM_SHARED((n, 128), jnp.float32),
            pltpu.SemaphoreType.DMA,
            pltpu.SemaphoreType.DMA,
            pltpu.SemaphoreType.DMA,
        ],
    )
    def body(hA_h, hB_h, cA_h, cB_h, pidx_h, oHA, oHB, oCA, oCB,
             pidx_v, rows0, rows1, rows2, src0, src1, src2,
             dst0, dst1, dst2, acc, s0, s1, s2):
        c = lax.axis_index("c")
        s = lax.axis_index("s")
        rows = (rows0, rows1, rows2)
        srcb = (src0, src1, src2)
        dstb = (dst0, dst1, dst2)
        sems = (s0, s1, s2)
        pltpu.sync_copy(pidx_h.at[s], pidx_v)
        stripe = pl.ds(s * rpt, rpt)

        def unpack(j, src_b, dst_b):
            for off in range(0, EK, 16):
                v = pidx_v[j, pl.ds(off, 16)]
                src_b[pl.ds(off, 16)] = v & 16383
                dst_b[pl.ds(off, 16)] = lax.shift_right_logical(v, 14)

        for tabA, tabB, outA, outB in ((hA_h, hB_h, oHA, oHB),
                                       (cA_h, cB_h, oCA, oCB)):
            _zero_fill(rows0, EK)
            _zero_acc_stripe(rows0, acc, s, rpt)
            plsc.subcore_barrier()

            def gather(src_b, buf, sem):
                @pl.when(c == 0)
                def _():
                    pltpu.async_copy(tabA.at[src_b], buf, sem)

                @pl.when(c == 1)
                def _():
                    pltpu.async_copy(tabB.at[src_b], buf, sem)

            def gwait(src_b, buf, sem):
                pltpu.make_async_copy(tabA.at[src_b], buf, sem).wait()

            for q in range(nbuf):
                unpack(q, srcb[q], dstb[q])
                gather(srcb[q], rows[q], sems[q])

            @pl.loop(0, trips)
            def _(p):
                for q in range(nbuf):
                    j = nbuf * p + q

                    @pl.when(j < nch_e)
                    def _():
                        gwait(srcb[q], rows[q], sems[q])
                        pltpu.sync_copy(rows[q], acc.at[dstb[q]], add=True)

                        @pl.when(j + nbuf < nch_e)
                        def _():
                            unpack(j + nbuf, srcb[q], dstb[q])
                            gather(srcb[q], rows[q], sems[q])

            plsc.subcore_barrier()

            @pl.when(c == 0)
            def _():
                pltpu.sync_copy(acc.at[stripe], outA.at[stripe])

            @pl.when(c == 1)
            def _():
                pltpu.sync_copy(acc.at[stripe], outB.at[stripe])

            plsc.subcore_barrier()

    return body(hA, hB, cA, cB, pidx_r)


def _sc_gather_rows(hA, hB, idx_r):
    n = hA.shape[0]
    nch = idx_r.shape[1]
    n_out = NS * nch * NK
    rpt = n_out // NS
    mesh = plsc.VectorSubcoreMesh(core_axis_name="c", subcore_axis_name="s")

    @functools.partial(
        pl.kernel,
        out_type=[jax.ShapeDtypeStruct((n_out, 128), jnp.float32)] * 2,
        mesh=mesh,
        scratch_types=[
            pltpu.VMEM((nch, NK), jnp.int32),
            pltpu.VMEM((NK, 128), jnp.float32),
            pltpu.SemaphoreType.DMA,
        ],
    )
    def body(hA_h, hB_h, idx_hbm, oA, oB, idx_v, rows_v, sem):
        c = lax.axis_index("c")
        s = lax.axis_index("s")
        pltpu.sync_copy(idx_hbm.at[s], idx_v)

        @pl.loop(0, nch)
        def _(j):
            @pl.when(c == 0)
            def _():
                pltpu.async_copy(hA_h.at[idx_v.at[j]], rows_v, sem).wait()
                pltpu.sync_copy(rows_v, oA.at[pl.ds(s * rpt + j * NK, NK)])

            @pl.when(c == 1)
            def _():
                pltpu.async_copy(hB_h.at[idx_v.at[j]], rows_v, sem).wait()
                pltpu.sync_copy(rows_v, oB.at[pl.ds(s * rpt + j * NK, NK)])

    return body(hA, hB, idx_r)


def _tc_wxh(xA, xB, Wx, b, bn):
    n = xA.shape[0]
    k, fo = Wx.shape

    def mm_body(xa_ref, xb_ref, w_ref, b_ref, o_ref):
        x = jnp.concatenate([xa_ref[...], xb_ref[...]], axis=1)
        o_ref[...] = (
            jnp.dot(x, w_ref[...], preferred_element_type=jnp.float32)
            + b_ref[...]
        )

    return pl.pallas_call(
        mm_body,
        grid=(n // bn,),
        in_specs=[
            pl.BlockSpec((bn, k // 2), lambda i: (i, 0)),
            pl.BlockSpec((bn, k // 2), lambda i: (i, 0)),
            pl.BlockSpec((k, fo), lambda i: (0, 0)),
            pl.BlockSpec((1, fo), lambda i: (0, 0)),
        ],
        out_specs=pl.BlockSpec((bn, fo), lambda i: (i, 0)),
        out_shape=jax.ShapeDtypeStruct((n, fo), jnp.float32),
    )(xA, xB, Wx, b.reshape(1, fo))


def _lstm_tail(g, c_agg, h_dim):
    i = g[:, 0:h_dim]
    f = g[:, h_dim:2 * h_dim]
    o = g[:, 2 * h_dim:3 * h_dim]
    gg = g[:, 3 * h_dim:4 * h_dim]
    cc = jax.nn.sigmoid(i) * jnp.tanh(gg)
    if c_agg is not None:
        cc += jax.nn.sigmoid(f) * c_agg
    hh = jax.nn.sigmoid(o) * jnp.tanh(cc)
    return hh, cc


def _tc_step0(wxh, h_dim, bn):
    n = wxh.shape[0]
    hh = h_dim // 2

    def body(wxh_ref, hA_o, hB_o, cA_o, cB_o):
        h, c = _lstm_tail(wxh_ref[...], None, h_dim)
        hA_o[...] = h[:, :hh]
        hB_o[...] = h[:, hh:]
        cA_o[...] = c[:, :hh]
        cB_o[...] = c[:, hh:]

    return pl.pallas_call(
        body,
        grid=(n // bn,),
        in_specs=[pl.BlockSpec((bn, 4 * h_dim), lambda i: (i, 0))],
        out_specs=[pl.BlockSpec((bn, hh), lambda i: (i, 0))] * 4,
        out_shape=[jax.ShapeDtypeStruct((n, hh), jnp.float32)] * 4,
    )(wxh)


def _tc_step(wxh, haA, haB, caA, caB, Wh, h_dim, bn):
    n = wxh.shape[0]
    hh = h_dim // 2

    def body(wxh_ref, haA_r, haB_r, caA_r, caB_r, wh_ref,
             hA_o, hB_o, cA_o, cB_o):
        h_agg = jnp.concatenate([haA_r[...], haB_r[...]], axis=1)
        g = wxh_ref[...] + jnp.dot(
            h_agg, wh_ref[...], preferred_element_type=jnp.float32
        )
        c_agg = jnp.concatenate([caA_r[...], caB_r[...]], axis=1)
        h, c = _lstm_tail(g, c_agg, h_dim)
        hA_o[...] = h[:, :hh]
        hB_o[...] = h[:, hh:]
        cA_o[...] = c[:, :hh]
        cB_o[...] = c[:, hh:]

    return pl.pallas_call(
        body,
        grid=(n // bn,),
        in_specs=[
            pl.BlockSpec((bn, 4 * h_dim), lambda i: (i, 0)),
            pl.BlockSpec((bn, hh), lambda i: (i, 0)),
            pl.BlockSpec((bn, hh), lambda i: (i, 0)),
            pl.BlockSpec((bn, hh), lambda i: (i, 0)),
            pl.BlockSpec((bn, hh), lambda i: (i, 0)),
            pl.BlockSpec((h_dim, 4 * h_dim), lambda i: (0, 0)),
        ],
        out_specs=[pl.BlockSpec((bn, hh), lambda i: (i, 0))] * 4,
        out_shape=[jax.ShapeDtypeStruct((n, hh), jnp.float32)] * 4,
    )(wxh, haA, haB, caA, caB, Wh)


def kernel(features, weights_x, weights_h, biases, index_map, edge_src,
           edge_dst):
    n, d = features.shape
    h_dim = weights_h.shape[1]
    n_layers = weights_h.shape[0]
    n_steps = 4
    bn = 512

    npad = NPAD
    pad_ids = jnp.arange(n, npad, dtype=jnp.int32)
    idx_p = jnp.concatenate([index_map.astype(jnp.int32), pad_ids])
    idx_r = idx_p.reshape(NS, npad // NS // NK, NK)
    e = edge_src.shape[0]
    pidx = edge_src.astype(jnp.int32) | (edge_dst.astype(jnp.int32) << 14)
    ept = e // NS
    ept_pad = -(-ept // EK) * EK
    pr = pidx.reshape(NS, ept)
    if ept_pad != ept:
        dummy = n + jnp.arange(NS, dtype=jnp.int32)
        dummy = (dummy | (dummy << 14))[:, None]
        pr = jnp.concatenate(
            [pr, jnp.broadcast_to(dummy, (NS, ept_pad - ept))], axis=1)
    pidx_r = pr.reshape(NS, ept_pad // EK, EK)

    zpad = jnp.zeros((npad - n, d // 2), jnp.float32)
    featA = jnp.concatenate([features[:, : d // 2], zpad])
    featB = jnp.concatenate([features[:, d // 2:], zpad])
    hA, hB = _sc_scatter_rows(featA, featB, idx_r, npad)

    start = 0
    for l in range(n_layers):
        in_dim = d if l == 0 else h_dim
        Wx = weights_x[start:start + in_dim]
        start += in_dim
        wxh = _tc_wxh(hA, hB, Wx, biases[l], bn)
        hA, hB, cA, cB = _tc_step0(wxh, h_dim, bn)
        for _ in range(n_steps - 1):
            haA, haB, caA, caB = _sc_segsum2(hA, hB, cA, cB, pidx_r)
            hA, hB, cA, cB = _tc_step(wxh, haA, haB, caA, caB,
                                      weights_h[l], h_dim, bn)

    oA, oB = _sc_gather_rows(hA, hB, idx_r)
    return jnp.concatenate([oA[:n], oB[:n]], axis=1)


# speedup vs baseline: 1.2643x; 1.0293x over previous
import functools

import jax
import jax.numpy as jnp
from jax import lax
from jax.experimental import pallas as pl
from jax.experimental.pallas import tpu as pltpu
from jax.experimental.pallas import tpu_sc as plsc

NS = 16
NC = 2
EK = 80
NK = 128
NPAD = NS * 5 * NK


def _zero_fill(buf, nrows):
    @pl.loop(0, nrows * 8)
    def _(t):
        buf[t // 8, pl.ds((t % 8) * 16, 16)] = jnp.zeros((16,), jnp.float32)


def _zero_acc_stripe(zbuf, acc, s, rows_per_tile):
    zr = zbuf.shape[0]
    nch = rows_per_tile // zr
    for k in range(nch):
        pltpu.sync_copy(zbuf, acc.at[pl.ds(s * rows_per_tile + k * zr, zr)])


def _sc_scatter_rows(featA, featB, idx_r, n_out):
    n = featA.shape[0]
    rpt_src = n // NS
    nch = rpt_src // NK
    rpt_out = n_out // NS
    mesh = plsc.VectorSubcoreMesh(core_axis_name="c", subcore_axis_name="s")

    @functools.partial(
        pl.kernel,
        out_type=[jax.ShapeDtypeStruct((n_out, 128), jnp.float32)] * 2,
        mesh=mesh,
        scratch_types=[
            pltpu.VMEM((nch, NK), jnp.int32),
            pltpu.VMEM((NK, 128), jnp.float32),
            pltpu.VMEM_SHARED((n_out, 128), jnp.float32),
            pltpu.SemaphoreType.DMA,
        ],
    )
    def body(fA, fB, idx_hbm, oA, oB, idx_v, rows_v, acc, sem):
        c = lax.axis_index("c")
        s = lax.axis_index("s")
        _zero_fill(rows_v, NK)
        _zero_acc_stripe(rows_v, acc, s, rpt_out)
        pltpu.sync_copy(idx_hbm.at[s], idx_v)
        plsc.subcore_barrier()

        @pl.loop(0, nch)
        def _(j):
            @pl.when(c == 0)
            def _():
                pltpu.async_copy(
                    fA.at[pl.ds(s * rpt_src + j * NK, NK)], rows_v, sem
                ).wait()

            @pl.when(c == 1)
            def _():
                pltpu.async_copy(
                    fB.at[pl.ds(s * rpt_src + j * NK, NK)], rows_v, sem
                ).wait()

            pltpu.sync_copy(rows_v, acc.at[idx_v.at[j]], add=True)

        plsc.subcore_barrier()
        stripe = pl.ds(s * rpt_out, rpt_out)

        @pl.when(c == 0)
        def _():
            pltpu.sync_copy(acc.at[stripe], oA.at[stripe])

        @pl.when(c == 1)
        def _():
            pltpu.sync_copy(acc.at[stripe], oB.at[stripe])

    return body(featA, featB, idx_r)


def _sc_segsum2(hA, hB, cA, cB, pidx_r):
    n = hA.shape[0]
    nch_e = pidx_r.shape[1]
    rpt = n // NS
    mesh = plsc.VectorSubcoreMesh(core_axis_name="c", subcore_axis_name="s")

    nbuf = 3
    trips = -(-nch_e // nbuf)

    @functools.partial(
        pl.kernel,
        out_type=[jax.ShapeDtypeStruct((n, 128), jnp.float32)] * 4,
        mesh=mesh,
        scratch_types=[
            pltpu.VMEM((nch_e, EK), jnp.int32),
            pltpu.VMEM((EK, 128), jnp.float32),
            pltpu.VMEM((EK, 128), jnp.float32),
            pltpu.VMEM((EK, 128), jnp.float32),
            pltpu.VMEM((EK,), jnp.int32),
            pltpu.VMEM((EK,), jnp.int32),
            pltpu.VMEM((EK,), jnp.int32),
            pltpu.VMEM((EK,), jnp.int32),
            pltpu.VMEM((EK,), jnp.int32),
            pltpu.VMEM((EK,), jnp.int32),
            pltpu.VMEM_SHARED((n, 128), jnp.float32),
            pltpu.SemaphoreType.DMA,
            pltpu.SemaphoreType.DMA,
            pltpu.SemaphoreType.DMA,
        ],
    )
    def body(hA_h, hB_h, cA_h, cB_h, pidx_h, oHA, oHB, oCA, oCB,
             pidx_v, rows0, rows1, rows2, src0, src1, src2,
             dst0, dst1, dst2, acc, s0, s1, s2):
        c = lax.axis_index("c")
        s = lax.axis_index("s")
        rows = (rows0, rows1, rows2)
        srcb = (src0, src1, src2)
        dstb = (dst0, dst1, dst2)
        sems = (s0, s1, s2)
        pltpu.sync_copy(pidx_h.at[s], pidx_v)
        stripe = pl.ds(s * rpt, rpt)

        def unpack(j, src_b, dst_b):
            for off in range(0, EK, 16):
                v = pidx_v[j, pl.ds(off, 16)]
                src_b[pl.ds(off, 16)] = v & 16383
                dst_b[pl.ds(off, 16)] = lax.shift_right_logical(v, 14)

        for tabA, tabB, outA, outB in ((hA_h, hB_h, oHA, oHB),
                                       (cA_h, cB_h, oCA, oCB)):
            _zero_fill(rows0, EK)
            _zero_acc_stripe(rows0, acc, s, rpt)
            plsc.subcore_barrier()

            def gather(src_b, buf, sem):
                @pl.when(c == 0)
                def _():
                    pltpu.async_copy(tabA.at[src_b], buf, sem)

                @pl.when(c == 1)
                def _():
                    pltpu.async_copy(tabB.at[src_b], buf, sem)

            def gwait(src_b, buf, sem):
                pltpu.make_async_copy(tabA.at[src_b], buf, sem).wait()

            for q in range(nbuf):
                unpack(q, srcb[q], dstb[q])
                gather(srcb[q], rows[q], sems[q])

            @pl.loop(0, trips)
            def _(p):
                for q in range(nbuf):
                    j = nbuf * p + q

                    @pl.when(j < nch_e)
                    def _():
                        gwait(srcb[q], rows[q], sems[q])
                        pltpu.sync_copy(rows[q], acc.at[dstb[q]], add=True)

                        @pl.when(j + nbuf < nch_e)
                        def _():
                            unpack(j + nbuf, srcb[q], dstb[q])
                            gather(srcb[q], rows[q], sems[q])

            plsc.subcore_barrier()

            @pl.when(c == 0)
            def _():
                pltpu.sync_copy(acc.at[stripe], outA.at[stripe])

            @pl.when(c == 1)
            def _():
                pltpu.sync_copy(acc.at[stripe], outB.at[stripe])

            plsc.subcore_barrier()

    return body(hA, hB, cA, cB, pidx_r)


def _sc_gather_rows(hA, hB, idx_r):
    n = hA.shape[0]
    nch = idx_r.shape[1]
    n_out = NS * nch * NK
    rpt = n_out // NS
    mesh = plsc.VectorSubcoreMesh(core_axis_name="c", subcore_axis_name="s")

    @functools.partial(
        pl.kernel,
        out_type=[jax.ShapeDtypeStruct((n_out, 128), jnp.float32)] * 2,
        mesh=mesh,
        scratch_types=[
            pltpu.VMEM((nch, NK), jnp.int32),
            pltpu.VMEM((NK, 128), jnp.float32),
            pltpu.SemaphoreType.DMA,
        ],
    )
    def body(hA_h, hB_h, idx_hbm, oA, oB, idx_v, rows_v, sem):
        c = lax.axis_index("c")
        s = lax.axis_index("s")
        pltpu.sync_copy(idx_hbm.at[s], idx_v)

        @pl.loop(0, nch)
        def _(j):
            @pl.when(c == 0)
            def _():
                pltpu.async_copy(hA_h.at[idx_v.at[j]], rows_v, sem).wait()
                pltpu.sync_copy(rows_v, oA.at[pl.ds(s * rpt + j * NK, NK)])

            @pl.when(c == 1)
            def _():
                pltpu.async_copy(hB_h.at[idx_v.at[j]], rows_v, sem).wait()
                pltpu.sync_copy(rows_v, oB.at[pl.ds(s * rpt + j * NK, NK)])

    return body(hA, hB, idx_r)


def _tc_wxh_step0(xA, xB, Wx, b, h_dim, bn):
    n = xA.shape[0]
    k, fo = Wx.shape
    hh = h_dim // 2

    def mm_body(xa_ref, xb_ref, w_ref, b_ref, o_ref,
                hA_o, hB_o, cA_o, cB_o):
        x = jnp.concatenate([xa_ref[...], xb_ref[...]], axis=1)
        wxh = (
            jnp.dot(x, w_ref[...], preferred_element_type=jnp.float32)
            + b_ref[...]
        )
        o_ref[...] = wxh
        h, c = _lstm_tail(wxh, None, h_dim)
        hA_o[...] = h[:, :hh]
        hB_o[...] = h[:, hh:]
        cA_o[...] = c[:, :hh]
        cB_o[...] = c[:, hh:]

    return pl.pallas_call(
        mm_body,
        grid=(n // bn,),
        in_specs=[
            pl.BlockSpec((bn, k // 2), lambda i: (i, 0)),
            pl.BlockSpec((bn, k // 2), lambda i: (i, 0)),
            pl.BlockSpec((k, fo), lambda i: (0, 0)),
            pl.BlockSpec((1, fo), lambda i: (0, 0)),
        ],
        out_specs=[pl.BlockSpec((bn, fo), lambda i: (i, 0))]
        + [pl.BlockSpec((bn, hh), lambda i: (i, 0))] * 4,
        out_shape=[jax.ShapeDtypeStruct((n, fo), jnp.float32)]
        + [jax.ShapeDtypeStruct((n, hh), jnp.float32)] * 4,
    )(xA, xB, Wx, b.reshape(1, fo))


def _lstm_tail(g, c_agg, h_dim):
    i = g[:, 0:h_dim]
    f = g[:, h_dim:2 * h_dim]
    o = g[:, 2 * h_dim:3 * h_dim]
    gg = g[:, 3 * h_dim:4 * h_dim]
    cc = jax.nn.sigmoid(i) * jnp.tanh(gg)
    if c_agg is not None:
        cc += jax.nn.sigmoid(f) * c_agg
    hh = jax.nn.sigmoid(o) * jnp.tanh(cc)
    return hh, cc


def _tc_step(wxh, haA, haB, caA, caB, Wh, h_dim, bn):
    n = wxh.shape[0]
    hh = h_dim // 2

    def body(wxh_ref, haA_r, haB_r, caA_r, caB_r, wh_ref,
             hA_o, hB_o, cA_o, cB_o):
        h_agg = jnp.concatenate([haA_r[...], haB_r[...]], axis=1)
        g = wxh_ref[...] + jnp.dot(
            h_agg, wh_ref[...], preferred_element_type=jnp.float32
        )
        c_agg = jnp.concatenate([caA_r[...], caB_r[...]], axis=1)
        h, c = _lstm_tail(g, c_agg, h_dim)
        hA_o[...] = h[:, :hh]
        hB_o[...] = h[:, hh:]
        cA_o[...] = c[:, :hh]
        cB_o[...] = c[:, hh:]

    return pl.pallas_call(
        body,
        grid=(n // bn,),
        in_specs=[
            pl.BlockSpec((bn, 4 * h_dim), lambda i: (i, 0)),
            pl.BlockSpec((bn, hh), lambda i: (i, 0)),
            pl.BlockSpec((bn, hh), lambda i: (i, 0)),
            pl.BlockSpec((bn, hh), lambda i: (i, 0)),
            pl.BlockSpec((bn, hh), lambda i: (i, 0)),
            pl.BlockSpec((h_dim, 4 * h_dim), lambda i: (0, 0)),
        ],
        out_specs=[pl.BlockSpec((bn, hh), lambda i: (i, 0))] * 4,
        out_shape=[jax.ShapeDtypeStruct((n, hh), jnp.float32)] * 4,
    )(wxh, haA, haB, caA, caB, Wh)


def kernel(features, weights_x, weights_h, biases, index_map, edge_src,
           edge_dst):
    n, d = features.shape
    h_dim = weights_h.shape[1]
    n_layers = weights_h.shape[0]
    n_steps = 4
    bn = 512

    npad = NPAD
    pad_ids = jnp.arange(n, npad, dtype=jnp.int32)
    idx_p = jnp.concatenate([index_map.astype(jnp.int32), pad_ids])
    idx_r = idx_p.reshape(NS, npad // NS // NK, NK)
    e = edge_src.shape[0]
    pidx = edge_src.astype(jnp.int32) | (edge_dst.astype(jnp.int32) << 14)
    ept = e // NS
    ept_pad = -(-ept // EK) * EK
    pr = pidx.reshape(NS, ept)
    if ept_pad != ept:
        dummy = n + jnp.arange(NS, dtype=jnp.int32)
        dummy = (dummy | (dummy << 14))[:, None]
        pr = jnp.concatenate(
            [pr, jnp.broadcast_to(dummy, (NS, ept_pad - ept))], axis=1)
    pidx_r = pr.reshape(NS, ept_pad // EK, EK)

    zpad = jnp.zeros((npad - n, d // 2), jnp.float32)
    featA = jnp.concatenate([features[:, : d // 2], zpad])
    featB = jnp.concatenate([features[:, d // 2:], zpad])
    hA, hB = _sc_scatter_rows(featA, featB, idx_r, npad)

    start = 0
    for l in range(n_layers):
        in_dim = d if l == 0 else h_dim
        Wx = weights_x[start:start + in_dim]
        start += in_dim
        wxh, hA, hB, cA, cB = _tc_wxh_step0(hA, hB, Wx, biases[l],
                                            h_dim, bn)
        for _ in range(n_steps - 1):
            haA, haB, caA, caB = _sc_segsum2(hA, hB, cA, cB, pidx_r)
            hA, hB, cA, cB = _tc_step(wxh, haA, haB, caA, caB,
                                      weights_h[l], h_dim, bn)

    oA, oB = _sc_gather_rows(hA, hB, idx_r)
    return jnp.concatenate([oA[:n], oB[:n]], axis=1)


# speedup vs baseline: 1.2961x; 1.0252x over previous
import functools

import jax
import jax.numpy as jnp
from jax import lax
from jax.experimental import pallas as pl
from jax.experimental.pallas import tpu as pltpu
from jax.experimental.pallas import tpu_sc as plsc

NS = 16
NC = 2
EK = 80
NK = 128
NPAD = NS * 5 * NK


def _zero_fill(buf, nrows):
    @pl.loop(0, nrows * 8)
    def _(t):
        buf[t // 8, pl.ds((t % 8) * 16, 16)] = jnp.zeros((16,), jnp.float32)


def _zero_acc_stripe(zbuf, acc, s, rows_per_tile):
    zr = zbuf.shape[0]
    nch = rows_per_tile // zr
    for k in range(nch):
        pltpu.sync_copy(zbuf, acc.at[pl.ds(s * rows_per_tile + k * zr, zr)])


def _sc_scatter_rows(featA, featB, idx_r, n_out):
    n = featA.shape[0]
    rpt_src = n // NS
    nch = rpt_src // NK
    rpt_out = n_out // NS
    mesh = plsc.VectorSubcoreMesh(core_axis_name="c", subcore_axis_name="s")

    @functools.partial(
        pl.kernel,
        out_type=[jax.ShapeDtypeStruct((n_out, 128), jnp.float32)] * 2,
        mesh=mesh,
        scratch_types=[
            pltpu.VMEM((nch, NK), jnp.int32),
            pltpu.VMEM((NK, 128), jnp.float32),
            pltpu.VMEM_SHARED((n_out, 128), jnp.float32),
            pltpu.SemaphoreType.DMA,
        ],
    )
    def body(fA, fB, idx_hbm, oA, oB, idx_v, rows_v, acc, sem):
        c = lax.axis_index("c")
        s = lax.axis_index("s")
        _zero_fill(rows_v, NK)
        _zero_acc_stripe(rows_v, acc, s, rpt_out)
        pltpu.sync_copy(idx_hbm.at[s], idx_v)
        plsc.subcore_barrier()

        @pl.loop(0, nch)
        def _(j):
            @pl.when(c == 0)
            def _():
                pltpu.async_copy(
                    fA.at[pl.ds(s * rpt_src + j * NK, NK)], rows_v, sem
                ).wait()

            @pl.when(c == 1)
            def _():
                pltpu.async_copy(
                    fB.at[pl.ds(s * rpt_src + j * NK, NK)], rows_v, sem
                ).wait()

            pltpu.sync_copy(rows_v, acc.at[idx_v.at[j]], add=True)

        plsc.subcore_barrier()
        stripe = pl.ds(s * rpt_out, rpt_out)

        @pl.when(c == 0)
        def _():
            pltpu.sync_copy(acc.at[stripe], oA.at[stripe])

        @pl.when(c == 1)
        def _():
            pltpu.sync_copy(acc.at[stripe], oB.at[stripe])

    return body(featA, featB, idx_r)


def _sc_segsum2(hA, hB, cA, cB, pidx_r):
    n = hA.shape[0]
    nch_e = pidx_r.shape[1]
    rpt = n // NS
    mesh = plsc.VectorSubcoreMesh(core_axis_name="c", subcore_axis_name="s")

    nbuf = 3
    trips = -(-nch_e // nbuf)

    @functools.partial(
        pl.kernel,
        out_type=[jax.ShapeDtypeStruct((n, 128), jnp.float32)] * 4,
        mesh=mesh,
        scratch_types=[
            pltpu.VMEM((nch_e, EK), jnp.int32),
            pltpu.VMEM((EK, 128), jnp.float32),
            pltpu.VMEM((EK, 128), jnp.float32),
            pltpu.VMEM((EK, 128), jnp.float32),
            pltpu.VMEM((EK,), jnp.int32),
            pltpu.VMEM((EK,), jnp.int32),
            pltpu.VMEM((EK,), jnp.int32),
            pltpu.VMEM((EK,), jnp.int32),
            pltpu.VMEM((EK,), jnp.int32),
            pltpu.VMEM((EK,), jnp.int32),
            pltpu.VMEM_SHARED((n, 128), jnp.float32),
            pltpu.SemaphoreType.DMA,
            pltpu.SemaphoreType.DMA,
            pltpu.SemaphoreType.DMA,
        ],
    )
    def body(hA_h, hB_h, cA_h, cB_h, pidx_h, oHA, oHB, oCA, oCB,
             pidx_v, rows0, rows1, rows2, src0, src1, src2,
             dst0, dst1, dst2, acc, s0, s1, s2):
        c = lax.axis_index("c")
        s = lax.axis_index("s")
        rows = (rows0, rows1, rows2)
        srcb = (src0, src1, src2)
        dstb = (dst0, dst1, dst2)
        sems = (s0, s1, s2)
        pltpu.sync_copy(pidx_h.at[s], pidx_v)
        stripe = pl.ds(s * rpt, rpt)

        def unpack(j, src_b, dst_b):
            for off in range(0, EK, 16):
                v = pidx_v[j, pl.ds(off, 16)]
                src_b[pl.ds(off, 16)] = v & 16383
                dst_b[pl.ds(off, 16)] = lax.shift_right_logical(v, 14)

        for tabA, tabB, outA, outB in ((hA_h, hB_h, oHA, oHB),
                                       (cA_h, cB_h, oCA, oCB)):
            _zero_fill(rows0, EK)
            _zero_acc_stripe(rows0, acc, s, rpt)
            plsc.subcore_barrier()

            def gather(src_b, buf, sem):
                @pl.when(c == 0)
                def _():
                    pltpu.async_copy(tabA.at[src_b], buf, sem)

                @pl.when(c == 1)
                def _():
                    pltpu.async_copy(tabB.at[src_b], buf, sem)

            def gwait(src_b, buf, sem):
                pltpu.make_async_copy(tabA.at[src_b], buf, sem).wait()

            for q in range(nbuf):
                unpack(q, srcb[q], dstb[q])
                gather(srcb[q], rows[q], sems[q])

            @pl.loop(0, trips)
            def _(p):
                for q in range(nbuf):
                    j = nbuf * p + q

                    @pl.when(j < nch_e)
                    def _():
                        gwait(srcb[q], rows[q], sems[q])
                        pltpu.sync_copy(rows[q], acc.at[dstb[q]], add=True)

                        @pl.when(j + nbuf < nch_e)
                        def _():
                            unpack(j + nbuf, srcb[q], dstb[q])
                            gather(srcb[q], rows[q], sems[q])

            plsc.subcore_barrier()

            @pl.when(c == 0)
            def _():
                pltpu.sync_copy(acc.at[stripe], outA.at[stripe])

            @pl.when(c == 1)
            def _():
                pltpu.sync_copy(acc.at[stripe], outB.at[stripe])

            plsc.subcore_barrier()

    return body(hA, hB, cA, cB, pidx_r)


def _sc_gather_rows(hA, hB, idx_r):
    n = hA.shape[0]
    nch = idx_r.shape[1]
    n_out = NS * nch * NK
    rpt = n_out // NS
    mesh = plsc.VectorSubcoreMesh(core_axis_name="c", subcore_axis_name="s")

    @functools.partial(
        pl.kernel,
        out_type=[jax.ShapeDtypeStruct((n_out, 128), jnp.float32)] * 2,
        mesh=mesh,
        scratch_types=[
            pltpu.VMEM((nch, NK), jnp.int32),
            pltpu.VMEM((NK, 128), jnp.float32),
            pltpu.SemaphoreType.DMA,
        ],
    )
    def body(hA_h, hB_h, idx_hbm, oA, oB, idx_v, rows_v, sem):
        c = lax.axis_index("c")
        s = lax.axis_index("s")
        pltpu.sync_copy(idx_hbm.at[s], idx_v)

        @pl.loop(0, nch)
        def _(j):
            @pl.when(c == 0)
            def _():
                pltpu.async_copy(hA_h.at[idx_v.at[j]], rows_v, sem).wait()
                pltpu.sync_copy(rows_v, oA.at[pl.ds(s * rpt + j * NK, NK)])

            @pl.when(c == 1)
            def _():
                pltpu.async_copy(hB_h.at[idx_v.at[j]], rows_v, sem).wait()
                pltpu.sync_copy(rows_v, oB.at[pl.ds(s * rpt + j * NK, NK)])

    return body(hA, hB, idx_r)


def _tc_wxh_step0(xA, xB, Wx, b, h_dim, bn):
    n = xA.shape[0]
    k, fo = Wx.shape
    hh = h_dim // 2

    def mm_body(xa_ref, xb_ref, w_ref, b_ref, o_ref,
                hA_o, hB_o, cA_o, cB_o):
        x = jnp.concatenate([xa_ref[...], xb_ref[...]], axis=1)
        wxh = (
            jnp.dot(x, w_ref[...], preferred_element_type=jnp.float32)
            + b_ref[...]
        )
        o_ref[...] = wxh
        h, c = _lstm_tail(wxh, None, h_dim)
        hA_o[...] = h[:, :hh]
        hB_o[...] = h[:, hh:]
        cA_o[...] = c[:, :hh]
        cB_o[...] = c[:, hh:]

    return pl.pallas_call(
        mm_body,
        grid=(n // bn,),
        in_specs=[
            pl.BlockSpec((bn, k // 2), lambda i: (i, 0)),
            pl.BlockSpec((bn, k // 2), lambda i: (i, 0)),
            pl.BlockSpec((k, fo), lambda i: (0, 0)),
            pl.BlockSpec((1, fo), lambda i: (0, 0)),
        ],
        out_specs=[pl.BlockSpec((bn, fo), lambda i: (i, 0))]
        + [pl.BlockSpec((bn, hh), lambda i: (i, 0))] * 4,
        out_shape=[jax.ShapeDtypeStruct((n, fo), jnp.float32)]
        + [jax.ShapeDtypeStruct((n, hh), jnp.float32)] * 4,
    )(xA, xB, Wx, b.reshape(1, fo))


def _lstm_tail(g, c_agg, h_dim):
    i = g[:, 0:h_dim]
    f = g[:, h_dim:2 * h_dim]
    o = g[:, 2 * h_dim:3 * h_dim]
    gg = g[:, 3 * h_dim:4 * h_dim]
    cc = jax.nn.sigmoid(i) * jnp.tanh(gg)
    if c_agg is not None:
        cc += jax.nn.sigmoid(f) * c_agg
    hh = jax.nn.sigmoid(o) * jnp.tanh(cc)
    return hh, cc


def _tc_step(wxh, haA, haB, caA, caB, Wh, h_dim, bn):
    n = wxh.shape[0]
    hh = h_dim // 2

    def body(wxh_ref, haA_r, haB_r, caA_r, caB_r, wh_ref,
             hA_o, hB_o, cA_o, cB_o):
        h_agg = jnp.concatenate([haA_r[...], haB_r[...]], axis=1)
        g = wxh_ref[...] + jnp.dot(
            h_agg, wh_ref[...], preferred_element_type=jnp.float32
        )
        c_agg = jnp.concatenate([caA_r[...], caB_r[...]], axis=1)
        h, c = _lstm_tail(g, c_agg, h_dim)
        hA_o[...] = h[:, :hh]
        hB_o[...] = h[:, hh:]
        cA_o[...] = c[:, :hh]
        cB_o[...] = c[:, hh:]

    return pl.pallas_call(
        body,
        grid=(n // bn,),
        in_specs=[
            pl.BlockSpec((bn, 4 * h_dim), lambda i: (i, 0)),
            pl.BlockSpec((bn, hh), lambda i: (i, 0)),
            pl.BlockSpec((bn, hh), lambda i: (i, 0)),
            pl.BlockSpec((bn, hh), lambda i: (i, 0)),
            pl.BlockSpec((bn, hh), lambda i: (i, 0)),
            pl.BlockSpec((h_dim, 4 * h_dim), lambda i: (0, 0)),
        ],
        out_specs=[pl.BlockSpec((bn, hh), lambda i: (i, 0))] * 4,
        out_shape=[jax.ShapeDtypeStruct((n, hh), jnp.float32)] * 4,
    )(wxh, haA, haB, caA, caB, Wh)


def kernel(features, weights_x, weights_h, biases, index_map, edge_src,
           edge_dst):
    n, d = features.shape
    h_dim = weights_h.shape[1]
    n_layers = weights_h.shape[0]
    n_steps = 4
    bn = 1024

    npad = NPAD
    pad_ids = jnp.arange(n, npad, dtype=jnp.int32)
    idx_p = jnp.concatenate([index_map.astype(jnp.int32), pad_ids])
    idx_r = idx_p.reshape(NS, npad // NS // NK, NK)
    e = edge_src.shape[0]
    pidx = edge_src.astype(jnp.int32) | (edge_dst.astype(jnp.int32) << 14)
    ept = e // NS
    ept_pad = -(-ept // EK) * EK
    pr = pidx.reshape(NS, ept)
    if ept_pad != ept:
        dummy = n + jnp.arange(NS, dtype=jnp.int32)
        dummy = (dummy | (dummy << 14))[:, None]
        pr = jnp.concatenate(
            [pr, jnp.broadcast_to(dummy, (NS, ept_pad - ept))], axis=1)
    pidx_r = pr.reshape(NS, ept_pad // EK, EK)

    zpad = jnp.zeros((npad - n, d // 2), jnp.float32)
    featA = jnp.concatenate([features[:, : d // 2], zpad])
    featB = jnp.concatenate([features[:, d // 2:], zpad])
    hA, hB = _sc_scatter_rows(featA, featB, idx_r, npad)

    start = 0
    for l in range(n_layers):
        in_dim = d if l == 0 else h_dim
        Wx = weights_x[start:start + in_dim]
        start += in_dim
        wxh, hA, hB, cA, cB = _tc_wxh_step0(hA, hB, Wx, biases[l],
                                            h_dim, bn)
        for _ in range(n_steps - 1):
            haA, haB, caA, caB = _sc_segsum2(hA, hB, cA, cB, pidx_r)
            hA, hB, cA, cB = _tc_step(wxh, haA, haB, caA, caB,
                                      weights_h[l], h_dim, bn)

    oA, oB = _sc_gather_rows(hA, hB, idx_r)
    return jnp.concatenate([oA[:n], oB[:n]], axis=1)


# speedup vs baseline: 1.2962x; 1.0000x over previous
import functools

import jax
import jax.numpy as jnp
from jax import lax
from jax.experimental import pallas as pl
from jax.experimental.pallas import tpu as pltpu
from jax.experimental.pallas import tpu_sc as plsc

NS = 16
NC = 2
EK = 80
NK = 128
NPAD = NS * 5 * NK


def _zero_fill(buf, nrows):
    @pl.loop(0, nrows * 8)
    def _(t):
        buf[t // 8, pl.ds((t % 8) * 16, 16)] = jnp.zeros((16,), jnp.float32)


def _zero_acc_stripe(zbuf, acc, s, rows_per_tile):
    zr = zbuf.shape[0]
    nch = rows_per_tile // zr
    for k in range(nch):
        pltpu.sync_copy(zbuf, acc.at[pl.ds(s * rows_per_tile + k * zr, zr)])


def _sc_scatter_rows(featA, featB, idx_r, n_out):
    n = featA.shape[0]
    rpt_src = n // NS
    nch = rpt_src // NK
    rpt_out = n_out // NS
    mesh = plsc.VectorSubcoreMesh(core_axis_name="c", subcore_axis_name="s")

    @functools.partial(
        pl.kernel,
        out_type=[jax.ShapeDtypeStruct((n_out, 128), jnp.float32)] * 2,
        mesh=mesh,
        scratch_types=[
            pltpu.VMEM((nch, NK), jnp.int32),
            pltpu.VMEM((NK, 128), jnp.float32),
            pltpu.VMEM_SHARED((n_out, 128), jnp.float32),
            pltpu.SemaphoreType.DMA,
        ],
    )
    def body(fA, fB, idx_hbm, oA, oB, idx_v, rows_v, acc, sem):
        c = lax.axis_index("c")
        s = lax.axis_index("s")
        _zero_fill(rows_v, NK)
        _zero_acc_stripe(rows_v, acc, s, rpt_out)
        pltpu.sync_copy(idx_hbm.at[s], idx_v)
        plsc.subcore_barrier()

        @pl.loop(0, nch)
        def _(j):
            @pl.when(c == 0)
            def _():
                pltpu.async_copy(
                    fA.at[pl.ds(s * rpt_src + j * NK, NK)], rows_v, sem
                ).wait()

            @pl.when(c == 1)
            def _():
                pltpu.async_copy(
                    fB.at[pl.ds(s * rpt_src + j * NK, NK)], rows_v, sem
                ).wait()

            pltpu.sync_copy(rows_v, acc.at[idx_v.at[j]], add=True)

        plsc.subcore_barrier()
        stripe = pl.ds(s * rpt_out, rpt_out)

        @pl.when(c == 0)
        def _():
            pltpu.sync_copy(acc.at[stripe], oA.at[stripe])

        @pl.when(c == 1)
        def _():
            pltpu.sync_copy(acc.at[stripe], oB.at[stripe])

    return body(featA, featB, idx_r)


def _sc_segsum2(hA, hB, cA, cB, pidx_r):
    n = hA.shape[0]
    nch_e = pidx_r.shape[1]
    rpt = n // NS
    mesh = plsc.VectorSubcoreMesh(core_axis_name="c", subcore_axis_name="s")

    nbuf = 3
    trips = -(-nch_e // nbuf)

    @functools.partial(
        pl.kernel,
        out_type=[jax.ShapeDtypeStruct((n, 128), jnp.float32)] * 4,
        mesh=mesh,
        scratch_types=[
            pltpu.VMEM((nch_e, EK), jnp.int32),
            pltpu.VMEM((EK, 128), jnp.float32),
            pltpu.VMEM((EK, 128), jnp.float32),
            pltpu.VMEM((EK, 128), jnp.float32),
            pltpu.VMEM((EK,), jnp.int32),
            pltpu.VMEM((EK,), jnp.int32),
            pltpu.VMEM((EK,), jnp.int32),
            pltpu.VMEM((EK,), jnp.int32),
            pltpu.VMEM((EK,), jnp.int32),
            pltpu.VMEM((EK,), jnp.int32),
            pltpu.VMEM_SHARED((n, 128), jnp.float32),
            pltpu.SemaphoreType.DMA,
            pltpu.SemaphoreType.DMA,
            pltpu.SemaphoreType.DMA,
        ],
    )
    def body(hA_h, hB_h, cA_h, cB_h, pidx_h, oHA, oHB, oCA, oCB,
             pidx_v, rows0, rows1, rows2, src0, src1, src2,
             dst0, dst1, dst2, acc, s0, s1, s2):
        c = lax.axis_index("c")
        s = lax.axis_index("s")
        rows = (rows0, rows1, rows2)
        srcb = (src0, src1, src2)
        dstb = (dst0, dst1, dst2)
        sems = (s0, s1, s2)
        pltpu.sync_copy(pidx_h.at[s], pidx_v)
        stripe = pl.ds(s * rpt, rpt)

        def unpack(j, src_b, dst_b):
            for off in range(0, EK, 16):
                v = pidx_v[j, pl.ds(off, 16)]
                src_b[pl.ds(off, 16)] = v & 16383
                dst_b[pl.ds(off, 16)] = lax.shift_right_logical(v, 14)

        for tabA, tabB, outA, outB in ((hA_h, hB_h, oHA, oHB),
                                       (cA_h, cB_h, oCA, oCB)):
            _zero_fill(rows0, EK)
            _zero_acc_stripe(rows0, acc, s, rpt)
            plsc.subcore_barrier()

            def gather(src_b, buf, sem):
                @pl.when(c == 0)
                def _():
                    pltpu.async_copy(tabA.at[src_b], buf, sem)

                @pl.when(c == 1)
                def _():
                    pltpu.async_copy(tabB.at[src_b], buf, sem)

            def gwait(src_b, buf, sem):
                pltpu.make_async_copy(tabA.at[src_b], buf, sem).wait()

            for q in range(nbuf):
                unpack(q, srcb[q], dstb[q])
                gather(srcb[q], rows[q], sems[q])

            @pl.loop(0, trips)
            def _(p):
                for q in range(nbuf):
                    j = nbuf * p + q

                    @pl.when(j < nch_e)
                    def _():
                        gwait(srcb[q], rows[q], sems[q])
                        pltpu.sync_copy(rows[q], acc.at[dstb[q]], add=True)

                        @pl.when(j + nbuf < nch_e)
                        def _():
                            unpack(j + nbuf, srcb[q], dstb[q])
                            gather(srcb[q], rows[q], sems[q])

            plsc.subcore_barrier()

            @pl.when(c == 0)
            def _():
                pltpu.sync_copy(acc.at[stripe], outA.at[stripe])

            @pl.when(c == 1)
            def _():
                pltpu.sync_copy(acc.at[stripe], outB.at[stripe])

            plsc.subcore_barrier()

    return body(hA, hB, cA, cB, pidx_r)


def _sc_gather_rows(hA, hB, idx_r):
    n = hA.shape[0]
    nch = idx_r.shape[1]
    n_out = NS * nch * NK
    rpt = n_out // NS
    mesh = plsc.VectorSubcoreMesh(core_axis_name="c", subcore_axis_name="s")

    @functools.partial(
        pl.kernel,
        out_type=[jax.ShapeDtypeStruct((n_out, 128), jnp.float32)] * 2,
        mesh=mesh,
        scratch_types=[
            pltpu.VMEM((nch, NK), jnp.int32),
            pltpu.VMEM((NK, 128), jnp.float32),
            pltpu.SemaphoreType.DMA,
        ],
    )
    def body(hA_h, hB_h, idx_hbm, oA, oB, idx_v, rows_v, sem):
        c = lax.axis_index("c")
        s = lax.axis_index("s")
        pltpu.sync_copy(idx_hbm.at[s], idx_v)

        @pl.loop(0, nch)
        def _(j):
            @pl.when(c == 0)
            def _():
                pltpu.async_copy(hA_h.at[idx_v.at[j]], rows_v, sem).wait()
                pltpu.sync_copy(rows_v, oA.at[pl.ds(s * rpt + j * NK, NK)])

            @pl.when(c == 1)
            def _():
                pltpu.async_copy(hB_h.at[idx_v.at[j]], rows_v, sem).wait()
                pltpu.sync_copy(rows_v, oB.at[pl.ds(s * rpt + j * NK, NK)])

    return body(hA, hB, idx_r)


def _tc_wxh_step0(xA, xB, Wx, b, h_dim, bn):
    n = xA.shape[0]
    k, fo = Wx.shape
    hh = h_dim // 2

    def mm_body(xa_ref, xb_ref, w_ref, b_ref, o_ref,
                hA_o, hB_o, cA_o, cB_o):
        x = jnp.concatenate([xa_ref[...], xb_ref[...]], axis=1)
        wxh = (
            jnp.dot(x, w_ref[...], preferred_element_type=jnp.float32)
            + b_ref[...]
        )
        o_ref[...] = wxh
        h, c = _lstm_tail(wxh, None, h_dim)
        hA_o[...] = h[:, :hh]
        hB_o[...] = h[:, hh:]
        cA_o[...] = c[:, :hh]
        cB_o[...] = c[:, hh:]

    return pl.pallas_call(
        mm_body,
        grid=(n // bn,),
        in_specs=[
            pl.BlockSpec((bn, k // 2), lambda i: (i, 0)),
            pl.BlockSpec((bn, k // 2), lambda i: (i, 0)),
            pl.BlockSpec((k, fo), lambda i: (0, 0)),
            pl.BlockSpec((1, fo), lambda i: (0, 0)),
        ],
        out_specs=[pl.BlockSpec((bn, fo), lambda i: (i, 0))]
        + [pl.BlockSpec((bn, hh), lambda i: (i, 0))] * 4,
        out_shape=[jax.ShapeDtypeStruct((n, fo), jnp.float32)]
        + [jax.ShapeDtypeStruct((n, hh), jnp.float32)] * 4,
    )(xA, xB, Wx, b.reshape(1, fo))


def _lstm_tail(g, c_agg, h_dim):
    i = g[:, 0:h_dim]
    f = g[:, h_dim:2 * h_dim]
    o = g[:, 2 * h_dim:3 * h_dim]
    gg = g[:, 3 * h_dim:4 * h_dim]
    cc = jax.nn.sigmoid(i) * jnp.tanh(gg)
    if c_agg is not None:
        cc += jax.nn.sigmoid(f) * c_agg
    hh = jax.nn.sigmoid(o) * jnp.tanh(cc)
    return hh, cc


def _tc_step(wxh, haA, haB, caA, caB, Wh, h_dim, bn):
    n = wxh.shape[0]
    hh = h_dim // 2

    def body(wxh_ref, haA_r, haB_r, caA_r, caB_r, wh_ref,
             hA_o, hB_o, cA_o, cB_o):
        h_agg = jnp.concatenate([haA_r[...], haB_r[...]], axis=1)
        g = wxh_ref[...] + jnp.dot(
            h_agg, wh_ref[...], preferred_element_type=jnp.float32
        )
        c_agg = jnp.concatenate([caA_r[...], caB_r[...]], axis=1)
        h, c = _lstm_tail(g, c_agg, h_dim)
        hA_o[...] = h[:, :hh]
        hB_o[...] = h[:, hh:]
        cA_o[...] = c[:, :hh]
        cB_o[...] = c[:, hh:]

    return pl.pallas_call(
        body,
        grid=(n // bn,),
        in_specs=[
            pl.BlockSpec((bn, 4 * h_dim), lambda i: (i, 0)),
            pl.BlockSpec((bn, hh), lambda i: (i, 0)),
            pl.BlockSpec((bn, hh), lambda i: (i, 0)),
            pl.BlockSpec((bn, hh), lambda i: (i, 0)),
            pl.BlockSpec((bn, hh), lambda i: (i, 0)),
            pl.BlockSpec((h_dim, 4 * h_dim), lambda i: (0, 0)),
        ],
        out_specs=[pl.BlockSpec((bn, hh), lambda i: (i, 0))] * 4,
        out_shape=[jax.ShapeDtypeStruct((n, hh), jnp.float32)] * 4,
    )(wxh, haA, haB, caA, caB, Wh)


def kernel(features, weights_x, weights_h, biases, index_map, edge_src,
           edge_dst):
    n, d = features.shape
    h_dim = weights_h.shape[1]
    n_layers = weights_h.shape[0]
    n_steps = 4
    bn = 2048

    npad = NPAD
    pad_ids = jnp.arange(n, npad, dtype=jnp.int32)
    idx_p = jnp.concatenate([index_map.astype(jnp.int32), pad_ids])
    idx_r = idx_p.reshape(NS, npad // NS // NK, NK)
    e = edge_src.shape[0]
    pidx = edge_src.astype(jnp.int32) | (edge_dst.astype(jnp.int32) << 14)
    ept = e // NS
    ept_pad = -(-ept // EK) * EK
    pr = pidx.reshape(NS, ept)
    if ept_pad != ept:
        dummy = n + jnp.arange(NS, dtype=jnp.int32)
        dummy = (dummy | (dummy << 14))[:, None]
        pr = jnp.concatenate(
            [pr, jnp.broadcast_to(dummy, (NS, ept_pad - ept))], axis=1)
    pidx_r = pr.reshape(NS, ept_pad // EK, EK)

    zpad = jnp.zeros((npad - n, d // 2), jnp.float32)
    featA = jnp.concatenate([features[:, : d // 2], zpad])
    featB = jnp.concatenate([features[:, d // 2:], zpad])
    hA, hB = _sc_scatter_rows(featA, featB, idx_r, npad)

    start = 0
    for l in range(n_layers):
        in_dim = d if l == 0 else h_dim
        Wx = weights_x[start:start + in_dim]
        start += in_dim
        wxh, hA, hB, cA, cB = _tc_wxh_step0(hA, hB, Wx, biases[l],
                                            h_dim, bn)
        for _ in range(n_steps - 1):
            haA, haB, caA, caB = _sc_segsum2(hA, hB, cA, cB, pidx_r)
            hA, hB, cA, cB = _tc_step(wxh, haA, haB, caA, caB,
                                      weights_h[l], h_dim, bn)

    oA, oB = _sc_gather_rows(hA, hB, idx_r)
    return jnp.concatenate([oA[:n], oB[:n]], axis=1)
